# baseline (speedup 1.0000x reference)
import functools

import jax
import jax.numpy as jnp
from jax.experimental import pallas as pl
from jax.experimental.pallas import tpu as pltpu


def _conv_pool_stats_kernel(w_ref, b_ref, x_ref, mx_ref, mn_ref, st_ref,
                            *, K, p, Cin, Cout):
    Ho, Wo = mx_ref.shape[2], mx_ref.shape[3]
    mx = [None] * Cout
    mn = [None] * Cout
    s1 = [None] * Cout
    s2 = [None] * Cout
    for pa in range(p):
        for pb in range(p):
            accs = [jnp.zeros((Ho, Wo), jnp.float32) for _ in range(Cout)]
            for kh in range(K):
                qh, al = divmod(pa + kh - 1, p)
                for kw in range(K):
                    qw, be = divmod(pb + kw - 1, p)
                    base = (al * p + be) * Cin
                    planes = x_ref[0, base:base + Cin,
                                   1 + qh:1 + qh + Ho, 1 + qw:1 + qw + Wo]
                    for ci in range(Cin):
                        pv = planes[ci]
                        for co in range(Cout):
                            w = w_ref[(co * Cin + ci) * K * K + kh * K + kw]
                            accs[co] = accs[co] + w * pv
            for co in range(Cout):
                y = jnp.maximum(accs[co] + b_ref[co], 0.0)
                ps1 = jnp.sum(y, axis=0)
                ps2 = jnp.sum(y * y, axis=0)
                if mx[co] is None:
                    mx[co], mn[co], s1[co], s2[co] = y, y, ps1, ps2
                else:
                    mx[co] = jnp.maximum(mx[co], y)
                    mn[co] = jnp.minimum(mn[co], y)
                    s1[co] = s1[co] + ps1
                    s2[co] = s2[co] + ps2
    for co in range(Cout):
        mx_ref[0, co] = mx[co]
        mn_ref[0, co] = mn[co]
        st_ref[0, co] = s1[co]
        st_ref[0, Cout + co] = s2[co]


def _bn_finalize_kernel(sc_ref, sh_ref, mx_ref, mn_ref, o_ref, *, Cout):
    for co in range(Cout):
        sc = sc_ref[co]
        sh = sh_ref[co]
        o_ref[0, co] = jnp.maximum(mx_ref[0, co] * sc, mn_ref[0, co] * sc) + sh


@functools.partial(jax.jit, static_argnames=("pool_size", "eps"))
def _forward(x_nchw, weight, bias, gamma, beta, *, pool_size=2, eps=1e-5):
    N, Cin, H, W = x_nchw.shape
    Cout, _, K, _ = weight.shape
    p = pool_size
    Ho, Wo = H // p, W // p

    x32 = x_nchw.astype(jnp.float32)
    xq = x32.reshape(N, Cin, Ho, p, Wo, p).transpose(0, 3, 5, 1, 2, 4)
    xpp = jnp.pad(xq, ((0, 0),) * 4 + ((1, 1), (1, 1)))
    xpp = xpp.reshape(N, p * p * Cin, Ho + 2, Wo + 2)

    wf = weight.astype(jnp.float32).reshape(Cout * Cin * K * K)
    bf = bias.astype(jnp.float32)

    kern = functools.partial(_conv_pool_stats_kernel, K=K, p=p, Cin=Cin, Cout=Cout)
    mx, mn, st = pl.pallas_call(
        kern,
        grid=(N,),
        in_specs=[
            pl.BlockSpec(memory_space=pltpu.SMEM),
            pl.BlockSpec(memory_space=pltpu.SMEM),
            pl.BlockSpec((1, p * p * Cin, Ho + 2, Wo + 2), lambda n: (n, 0, 0, 0)),
        ],
        out_specs=(
            pl.BlockSpec((1, Cout, Ho, Wo), lambda n: (n, 0, 0, 0)),
            pl.BlockSpec((1, Cout, Ho, Wo), lambda n: (n, 0, 0, 0)),
            pl.BlockSpec((1, 2 * Cout, Wo), lambda n: (n, 0, 0)),
        ),
        out_shape=(
            jax.ShapeDtypeStruct((N, Cout, Ho, Wo), jnp.float32),
            jax.ShapeDtypeStruct((N, Cout, Ho, Wo), jnp.float32),
            jax.ShapeDtypeStruct((N, 2 * Cout, Wo), jnp.float32),
        ),
        compiler_params=pltpu.CompilerParams(
            dimension_semantics=("parallel",)),
    )(wf, bf, xpp)

    s = jnp.sum(st, axis=(0, 2))
    cnt = float(N * H * W)
    mean = s[:Cout] / cnt
    var = jnp.maximum(s[Cout:] / cnt - mean * mean, 0.0)
    scale = gamma.astype(jnp.float32) * jax.lax.rsqrt(var + eps)
    shift = beta.astype(jnp.float32) - mean * scale

    out = pl.pallas_call(
        functools.partial(_bn_finalize_kernel, Cout=Cout),
        grid=(N,),
        in_specs=[
            pl.BlockSpec(memory_space=pltpu.SMEM),
            pl.BlockSpec(memory_space=pltpu.SMEM),
            pl.BlockSpec((1, Cout, Ho, Wo), lambda n: (n, 0, 0, 0)),
            pl.BlockSpec((1, Cout, Ho, Wo), lambda n: (n, 0, 0, 0)),
        ],
        out_specs=pl.BlockSpec((1, Cout, Ho, Wo), lambda n: (n, 0, 0, 0)),
        out_shape=jax.ShapeDtypeStruct((N, Cout, Ho, Wo), jnp.float32),
        compiler_params=pltpu.CompilerParams(
            dimension_semantics=("parallel",)),
    )(scale, shift, mx, mn)

    return out


def kernel(x_nchw, weight, bias, gamma, beta):
    return _forward(x_nchw, weight, bias, gamma, beta, pool_size=2)


# speedup vs baseline: 1.8963x; 1.8963x over previous
import functools

import jax
import jax.numpy as jnp
from jax.experimental import pallas as pl
from jax.experimental.pallas import tpu as pltpu


def _lane_shift(v, d):
    if d == 0:
        return v
    C, Hh, W = v.shape
    vp = jnp.pad(v, ((0, 0), (0, 0), (1, 1)))
    return vp[:, :, 1 + d:1 + d + W]


def _row_up(v):
    C, Hh, W = v.shape
    return jnp.pad(v, ((0, 0), (1, 0), (0, 0)))[:, :Hh, :]


def _row_down(v):
    C, Hh, W = v.shape
    return jnp.pad(v, ((0, 0), (0, 1), (0, 0)))[:, 1:, :]


def _conv_pool_stats_kernel(w_ref, b_ref, x_ref, mx_ref, mn_ref, st_ref,
                            *, K, Cin, Cout):
    Ho, W = x_ref.shape[2], x_ref.shape[3]
    Wo = W // 2
    ev = x_ref[0, 0:Cin]
    od = x_ref[0, Cin:2 * Cin]

    BE = [_lane_shift(ev, d) for d in (-1, 0, 1)]
    BO = [_lane_shift(od, d) for d in (-1, 0, 1)]

    views = {
        -1: [_row_up(b) for b in BO],
        0: BE,
        1: BO,
        2: [_row_down(b) for b in BE],
    }

    mxs = [None] * Cout
    mns = [None] * Cout
    s1 = [None] * Cout
    s2 = [None] * Cout
    for pa in range(2):
        for co in range(Cout):
            terms = []
            for kh in range(K):
                vg = views[pa + kh - 1]
                for kw in range(K):
                    v = vg[kw]
                    for ci in range(Cin):
                        w = w_ref[(co * Cin + ci) * K * K + kh * K + kw]
                        terms.append(w * v[ci])
            acc = terms[0] + b_ref[co]
            for t in terms[1:]:
                acc = acc + t
            y = jnp.maximum(acc, 0.0)
            ysh = jnp.pad(y, ((0, 0), (0, 1)))[:, 1:]
            tmx = jnp.maximum(y, ysh)
            tmn = jnp.minimum(y, ysh)
            ps1 = jnp.sum(y, axis=0)
            ps2 = jnp.sum(y * y, axis=0)
            if mxs[co] is None:
                mxs[co], mns[co], s1[co], s2[co] = tmx, tmn, ps1, ps2
            else:
                mxs[co] = jnp.maximum(mxs[co], tmx)
                mns[co] = jnp.minimum(mns[co], tmn)
                s1[co] = s1[co] + ps1
                s2[co] = s2[co] + ps2

    rr = jax.lax.broadcasted_iota(jnp.int32, (W, Wo), 0)
    cc = jax.lax.broadcasted_iota(jnp.int32, (W, Wo), 1)
    S = (rr == 2 * cc).astype(jnp.float32)
    for co in range(Cout):
        mx_ref[0, co] = jnp.dot(mxs[co], S, preferred_element_type=jnp.float32)
        mn_ref[0, co] = jnp.dot(mns[co], S, preferred_element_type=jnp.float32)
        st_ref[0, co] = s1[co]
        st_ref[0, Cout + co] = s2[co]


def _bn_finalize_kernel(sc_ref, sh_ref, mx_ref, mn_ref, o_ref, *, Cout):
    for co in range(Cout):
        sc = sc_ref[co]
        sh = sh_ref[co]
        o_ref[0, co] = jnp.maximum(mx_ref[0, co] * sc, mn_ref[0, co] * sc) + sh


@functools.partial(jax.jit, static_argnames=("pool_size", "eps"))
def _forward(x_nchw, weight, bias, gamma, beta, *, pool_size=2, eps=1e-5):
    N, Cin, H, W = x_nchw.shape
    Cout, _, K, _ = weight.shape
    p = pool_size
    Ho, Wo = H // p, W // p

    x32 = x_nchw.astype(jnp.float32)
    xhp = x32.reshape(N, Cin, Ho, p, W).transpose(0, 3, 1, 2, 4)
    xhp = xhp.reshape(N, p * Cin, Ho, W)

    wf = weight.astype(jnp.float32).reshape(Cout * Cin * K * K)
    bf = bias.astype(jnp.float32)

    kern = functools.partial(_conv_pool_stats_kernel, K=K, Cin=Cin, Cout=Cout)
    mx, mn, st = pl.pallas_call(
        kern,
        grid=(N,),
        in_specs=[
            pl.BlockSpec(memory_space=pltpu.SMEM),
            pl.BlockSpec(memory_space=pltpu.SMEM),
            pl.BlockSpec((1, p * Cin, Ho, W), lambda n: (n, 0, 0, 0)),
        ],
        out_specs=(
            pl.BlockSpec((1, Cout, Ho, Wo), lambda n: (n, 0, 0, 0)),
            pl.BlockSpec((1, Cout, Ho, Wo), lambda n: (n, 0, 0, 0)),
            pl.BlockSpec((1, 2 * Cout, W), lambda n: (n, 0, 0)),
        ),
        out_shape=(
            jax.ShapeDtypeStruct((N, Cout, Ho, Wo), jnp.float32),
            jax.ShapeDtypeStruct((N, Cout, Ho, Wo), jnp.float32),
            jax.ShapeDtypeStruct((N, 2 * Cout, W), jnp.float32),
        ),
        compiler_params=pltpu.CompilerParams(
            dimension_semantics=("parallel",)),
    )(wf, bf, xhp)

    s = jnp.sum(st, axis=(0, 2))
    cnt = float(N * H * W)
    mean = s[:Cout] / cnt
    var = jnp.maximum(s[Cout:] / cnt - mean * mean, 0.0)
    scale = gamma.astype(jnp.float32) * jax.lax.rsqrt(var + eps)
    shift = beta.astype(jnp.float32) - mean * scale

    out = pl.pallas_call(
        functools.partial(_bn_finalize_kernel, Cout=Cout),
        grid=(N,),
        in_specs=[
            pl.BlockSpec(memory_space=pltpu.SMEM),
            pl.BlockSpec(memory_space=pltpu.SMEM),
            pl.BlockSpec((1, Cout, Ho, Wo), lambda n: (n, 0, 0, 0)),
            pl.BlockSpec((1, Cout, Ho, Wo), lambda n: (n, 0, 0, 0)),
        ],
        out_specs=pl.BlockSpec((1, Cout, Ho, Wo), lambda n: (n, 0, 0, 0)),
        out_shape=jax.ShapeDtypeStruct((N, Cout, Ho, Wo), jnp.float32),
        compiler_params=pltpu.CompilerParams(
            dimension_semantics=("parallel",)),
    )(scale, shift, mx, mn)

    return out


def kernel(x_nchw, weight, bias, gamma, beta):
    return _forward(x_nchw, weight, bias, gamma, beta, pool_size=2)


# speedup vs baseline: 1.9322x; 1.0189x over previous
import functools

import jax
import jax.numpy as jnp
from jax.experimental import pallas as pl
from jax.experimental.pallas import tpu as pltpu


def _lane_shift(v, d):
    if d == 0:
        return v
    C, Hh, W = v.shape
    vp = jnp.pad(v, ((0, 0), (0, 0), (1, 1)))
    return vp[:, :, 1 + d:1 + d + W]


def _row_up(v):
    C, Hh, W = v.shape
    return jnp.pad(v, ((0, 0), (1, 0), (0, 0)))[:, :Hh, :]


def _row_down(v):
    C, Hh, W = v.shape
    return jnp.pad(v, ((0, 0), (0, 1), (0, 0)))[:, 1:, :]


def _conv_pool_stats_kernel(w_ref, b_ref, x_ref, mx_ref, mn_ref, st_ref,
                            *, K, Cin, Cout):
    Ho, W = x_ref.shape[2], x_ref.shape[3]
    Wo = W // 2
    ev = x_ref[0, 0:Cin]
    od = x_ref[0, Cin:2 * Cin]

    BE = [_lane_shift(ev, d) for d in (-1, 0, 1)]
    BO = [_lane_shift(od, d) for d in (-1, 0, 1)]

    views = {
        -1: [_row_up(b) for b in BO],
        0: BE,
        1: BO,
        2: [_row_down(b) for b in BE],
    }

    mxs = [None] * Cout
    mns = [None] * Cout
    s1 = [None] * Cout
    s2 = [None] * Cout
    for pa in range(2):
        accs = [jnp.full((Ho, W), b_ref[co], jnp.float32) for co in range(Cout)]
        for kh in range(K):
            vg = views[pa + kh - 1]
            for kw in range(K):
                v = vg[kw]
                for ci in range(Cin):
                    pv = v[ci]
                    for co in range(Cout):
                        w = w_ref[(co * Cin + ci) * K * K + kh * K + kw]
                        accs[co] = accs[co] + w * pv
        for co in range(Cout):
            y = jnp.maximum(accs[co], 0.0)
            ysh = jnp.pad(y, ((0, 0), (0, 1)))[:, 1:]
            tmx = jnp.maximum(y, ysh)
            tmn = jnp.minimum(y, ysh)
            ps1 = jnp.sum(y, axis=0)
            ps2 = jnp.sum(y * y, axis=0)
            if mxs[co] is None:
                mxs[co], mns[co], s1[co], s2[co] = tmx, tmn, ps1, ps2
            else:
                mxs[co] = jnp.maximum(mxs[co], tmx)
                mns[co] = jnp.minimum(mns[co], tmn)
                s1[co] = s1[co] + ps1
                s2[co] = s2[co] + ps2

    rr = jax.lax.broadcasted_iota(jnp.int32, (W, Wo), 0)
    cc = jax.lax.broadcasted_iota(jnp.int32, (W, Wo), 1)
    S = (rr == 2 * cc).astype(jnp.float32)
    for co in range(Cout):
        mx_ref[0, co] = jnp.dot(mxs[co], S, preferred_element_type=jnp.float32)
        mn_ref[0, co] = jnp.dot(mns[co], S, preferred_element_type=jnp.float32)
        st_ref[0, co] = s1[co]
        st_ref[0, Cout + co] = s2[co]


def _bn_finalize_kernel(sc_ref, sh_ref, mx_ref, mn_ref, o_ref, *, Cout):
    for co in range(Cout):
        sc = sc_ref[co]
        sh = sh_ref[co]
        o_ref[0, co] = jnp.maximum(mx_ref[0, co] * sc, mn_ref[0, co] * sc) + sh


@functools.partial(jax.jit, static_argnames=("pool_size", "eps"))
def _forward(x_nchw, weight, bias, gamma, beta, *, pool_size=2, eps=1e-5):
    N, Cin, H, W = x_nchw.shape
    Cout, _, K, _ = weight.shape
    p = pool_size
    Ho, Wo = H // p, W // p

    x32 = x_nchw.astype(jnp.float32)
    xhp = x32.reshape(N, Cin, Ho, p, W).transpose(0, 3, 1, 2, 4)
    xhp = xhp.reshape(N, p * Cin, Ho, W)

    wf = weight.astype(jnp.float32).reshape(Cout * Cin * K * K)
    bf = bias.astype(jnp.float32)

    kern = functools.partial(_conv_pool_stats_kernel, K=K, Cin=Cin, Cout=Cout)
    mx, mn, st = pl.pallas_call(
        kern,
        grid=(N,),
        in_specs=[
            pl.BlockSpec(memory_space=pltpu.SMEM),
            pl.BlockSpec(memory_space=pltpu.SMEM),
            pl.BlockSpec((1, p * Cin, Ho, W), lambda n: (n, 0, 0, 0)),
        ],
        out_specs=(
            pl.BlockSpec((1, Cout, Ho, Wo), lambda n: (n, 0, 0, 0)),
            pl.BlockSpec((1, Cout, Ho, Wo), lambda n: (n, 0, 0, 0)),
            pl.BlockSpec((1, 2 * Cout, W), lambda n: (n, 0, 0)),
        ),
        out_shape=(
            jax.ShapeDtypeStruct((N, Cout, Ho, Wo), jnp.float32),
            jax.ShapeDtypeStruct((N, Cout, Ho, Wo), jnp.float32),
            jax.ShapeDtypeStruct((N, 2 * Cout, W), jnp.float32),
        ),
        compiler_params=pltpu.CompilerParams(
            dimension_semantics=("parallel",)),
    )(wf, bf, xhp)

    s = jnp.sum(st, axis=(0, 2))
    cnt = float(N * H * W)
    mean = s[:Cout] / cnt
    var = jnp.maximum(s[Cout:] / cnt - mean * mean, 0.0)
    scale = gamma.astype(jnp.float32) * jax.lax.rsqrt(var + eps)
    shift = beta.astype(jnp.float32) - mean * scale

    out = pl.pallas_call(
        functools.partial(_bn_finalize_kernel, Cout=Cout),
        grid=(N,),
        in_specs=[
            pl.BlockSpec(memory_space=pltpu.SMEM),
            pl.BlockSpec(memory_space=pltpu.SMEM),
            pl.BlockSpec((1, Cout, Ho, Wo), lambda n: (n, 0, 0, 0)),
            pl.BlockSpec((1, Cout, Ho, Wo), lambda n: (n, 0, 0, 0)),
        ],
        out_specs=pl.BlockSpec((1, Cout, Ho, Wo), lambda n: (n, 0, 0, 0)),
        out_shape=jax.ShapeDtypeStruct((N, Cout, Ho, Wo), jnp.float32),
        compiler_params=pltpu.CompilerParams(
            dimension_semantics=("parallel",)),
    )(scale, shift, mx, mn)

    return out


def kernel(x_nchw, weight, bias, gamma, beta):
    return _forward(x_nchw, weight, bias, gamma, beta, pool_size=2)


# speedup vs baseline: 1.9441x; 1.0062x over previous
import functools

import jax
import jax.numpy as jnp
from jax.experimental import pallas as pl
from jax.experimental.pallas import tpu as pltpu


def _lane_shift(v, d):
    if d == 0:
        return v
    C, Hh, W = v.shape
    vp = jnp.pad(v, ((0, 0), (0, 0), (1, 1)))
    return vp[:, :, 1 + d:1 + d + W]


def _row_up(v):
    C, Hh, W = v.shape
    return jnp.pad(v, ((0, 0), (1, 0), (0, 0)))[:, :Hh, :]


def _row_down(v):
    C, Hh, W = v.shape
    return jnp.pad(v, ((0, 0), (0, 1), (0, 0)))[:, 1:, :]


def _conv_pool_stats_kernel(w_ref, b_ref, x_ref, mx_ref, mn_ref, st_ref,
                            *, K, Cin, Cout):
    Ho, W = x_ref.shape[2], x_ref.shape[3]
    Wo = W // 2
    ev = x_ref[0, 0:Cin]
    od = x_ref[0, Cin:2 * Cin]

    BE = [_lane_shift(ev, d) for d in (-1, 0, 1)]
    BO = [_lane_shift(od, d) for d in (-1, 0, 1)]

    views = {
        -1: [_row_up(b) for b in BO],
        0: BE,
        1: BO,
        2: [_row_down(b) for b in BE],
    }

    rr = jax.lax.broadcasted_iota(jnp.int32, (W, Wo), 0)
    cc = jax.lax.broadcasted_iota(jnp.int32, (W, Wo), 1)
    s_even = (rr == 2 * cc).astype(jnp.float32)
    s_odd = (rr == 2 * cc + 1).astype(jnp.float32)

    mxs = [None] * Cout
    mns = [None] * Cout
    s1 = [None] * Cout
    s2 = [None] * Cout
    for pa in range(2):
        accs = [jnp.full((Ho, W), b_ref[co], jnp.float32) for co in range(Cout)]
        for kh in range(K):
            vg = views[pa + kh - 1]
            for kw in range(K):
                v = vg[kw]
                for ci in range(Cin):
                    pv = v[ci]
                    for co in range(Cout):
                        w = w_ref[(co * Cin + ci) * K * K + kh * K + kw]
                        accs[co] = accs[co] + w * pv
        for co in range(Cout):
            y = jnp.maximum(accs[co], 0.0)
            ye = jnp.dot(y, s_even, preferred_element_type=jnp.float32)
            yo = jnp.dot(y, s_odd, preferred_element_type=jnp.float32)
            ps1 = jnp.sum(y, axis=0)
            ps2 = jnp.sum(y * y, axis=0)
            if mxs[co] is None:
                mxs[co] = jnp.maximum(ye, yo)
                mns[co] = jnp.minimum(ye, yo)
                s1[co], s2[co] = ps1, ps2
            else:
                mxs[co] = jnp.maximum(mxs[co], jnp.maximum(ye, yo))
                mns[co] = jnp.minimum(mns[co], jnp.minimum(ye, yo))
                s1[co] = s1[co] + ps1
                s2[co] = s2[co] + ps2

    for co in range(Cout):
        mx_ref[0, co] = mxs[co]
        mn_ref[0, co] = mns[co]
        st_ref[0, co] = s1[co]
        st_ref[0, Cout + co] = s2[co]


def _bn_finalize_kernel(sc_ref, sh_ref, mx_ref, mn_ref, o_ref, *, Cout):
    for co in range(Cout):
        sc = sc_ref[co]
        sh = sh_ref[co]
        o_ref[0, co] = jnp.maximum(mx_ref[0, co] * sc, mn_ref[0, co] * sc) + sh


@functools.partial(jax.jit, static_argnames=("pool_size", "eps"))
def _forward(x_nchw, weight, bias, gamma, beta, *, pool_size=2, eps=1e-5):
    N, Cin, H, W = x_nchw.shape
    Cout, _, K, _ = weight.shape
    p = pool_size
    Ho, Wo = H // p, W // p

    x32 = x_nchw.astype(jnp.float32)
    xhp = x32.reshape(N, Cin, Ho, p, W).transpose(0, 3, 1, 2, 4)
    xhp = xhp.reshape(N, p * Cin, Ho, W)

    wf = weight.astype(jnp.float32).reshape(Cout * Cin * K * K)
    bf = bias.astype(jnp.float32)

    kern = functools.partial(_conv_pool_stats_kernel, K=K, Cin=Cin, Cout=Cout)
    mx, mn, st = pl.pallas_call(
        kern,
        grid=(N,),
        in_specs=[
            pl.BlockSpec(memory_space=pltpu.SMEM),
            pl.BlockSpec(memory_space=pltpu.SMEM),
            pl.BlockSpec((1, p * Cin, Ho, W), lambda n: (n, 0, 0, 0)),
        ],
        out_specs=(
            pl.BlockSpec((1, Cout, Ho, Wo), lambda n: (n, 0, 0, 0)),
            pl.BlockSpec((1, Cout, Ho, Wo), lambda n: (n, 0, 0, 0)),
            pl.BlockSpec((1, 2 * Cout, W), lambda n: (n, 0, 0)),
        ),
        out_shape=(
            jax.ShapeDtypeStruct((N, Cout, Ho, Wo), jnp.float32),
            jax.ShapeDtypeStruct((N, Cout, Ho, Wo), jnp.float32),
            jax.ShapeDtypeStruct((N, 2 * Cout, W), jnp.float32),
        ),
        compiler_params=pltpu.CompilerParams(
            dimension_semantics=("parallel",)),
    )(wf, bf, xhp)

    s = jnp.sum(st, axis=(0, 2))
    cnt = float(N * H * W)
    mean = s[:Cout] / cnt
    var = jnp.maximum(s[Cout:] / cnt - mean * mean, 0.0)
    scale = gamma.astype(jnp.float32) * jax.lax.rsqrt(var + eps)
    shift = beta.astype(jnp.float32) - mean * scale

    out = pl.pallas_call(
        functools.partial(_bn_finalize_kernel, Cout=Cout),
        grid=(N,),
        in_specs=[
            pl.BlockSpec(memory_space=pltpu.SMEM),
            pl.BlockSpec(memory_space=pltpu.SMEM),
            pl.BlockSpec((1, Cout, Ho, Wo), lambda n: (n, 0, 0, 0)),
            pl.BlockSpec((1, Cout, Ho, Wo), lambda n: (n, 0, 0, 0)),
        ],
        out_specs=pl.BlockSpec((1, Cout, Ho, Wo), lambda n: (n, 0, 0, 0)),
        out_shape=jax.ShapeDtypeStruct((N, Cout, Ho, Wo), jnp.float32),
        compiler_params=pltpu.CompilerParams(
            dimension_semantics=("parallel",)),
    )(scale, shift, mx, mn)

    return out


def kernel(x_nchw, weight, bias, gamma, beta):
    return _forward(x_nchw, weight, bias, gamma, beta, pool_size=2)
